# baseline (speedup 1.0000x reference)
import functools

import jax
import jax.numpy as jnp
from jax import lax
from jax.experimental import pallas as pl
from jax.experimental.pallas import tpu as pltpu
from jax.experimental.pallas import tpu_sc as plsc


def _emb_call(N, D, NC, NS):
    NW = NC * NS
    n_per_w = N // NW
    C = 128
    K = 8
    BLK = K * C
    NBUF = 3
    n_blk = n_per_w // BLK
    assert n_per_w % BLK == 0 and n_blk >= NBUF + 1

    mesh = plsc.VectorSubcoreMesh(core_axis_name="c", subcore_axis_name="s",
                                  num_cores=NC, num_subcores=NS)

    @functools.partial(
        pl.kernel,
        out_type=jax.ShapeDtypeStruct((N, D), jnp.float32),
        mesh=mesh,
        scratch_types=[
            pltpu.VMEM((NBUF, K, C), jnp.int32),
            pltpu.VMEM((NBUF, BLK, D), jnp.float32),
            [pltpu.SemaphoreType.DMA] * NBUF,
            [pltpu.SemaphoreType.DMA] * NBUF,
        ],
        compiler_params=pltpu.CompilerParams(use_tc_tiling_on_sc=False),
    )
    def emb(idx_hbm, table_hbm, out_hbm, idx_v, rows_v, gsems, osems):
        wid = lax.axis_index("s") * NC + lax.axis_index("c")
        base = wid * n_per_w

        def fire(i, p):
            b = pl.multiple_of(base + i * BLK, BLK)
            pltpu.sync_copy(idx_hbm.at[pl.ds(pl.multiple_of(b // C, K), K)],
                            idx_v.at[p])
            for j in range(K):
                pltpu.async_copy(table_hbm.at[idx_v.at[p, j]],
                                 rows_v.at[p, pl.ds(j * C, C)], gsems[p])

        def drain_gathers(p):
            pltpu.make_async_copy(out_hbm.at[pl.ds(0, BLK)], rows_v.at[p],
                                  gsems[p]).wait()

        def start_store(i, p):
            b = pl.multiple_of(base + i * BLK, BLK)
            pltpu.async_copy(rows_v.at[p], out_hbm.at[pl.ds(b, BLK)],
                             osems[p])

        def wait_store(p):
            pltpu.make_async_copy(out_hbm.at[pl.ds(0, BLK)], rows_v.at[p],
                                  osems[p]).wait()

        fire(0, 0)
        for i in range(1, NBUF):
            fire(i, i)
            drain_gathers(i - 1)
            start_store(i - 1, i - 1)

        n_loop = (n_blk - NBUF) // NBUF

        def body(g, carry):
            for t in range(NBUF):
                i = g * NBUF + t
                wait_store(t)
                fire(i, t)
                q = (t + NBUF - 1) % NBUF
                drain_gathers(q)
                start_store(i - 1, q)
            return carry

        lax.fori_loop(1, n_loop + 1, body, 0)

        for r in range(NBUF * (n_loop + 1), n_blk):
            t = r % NBUF
            wait_store(t)
            fire(r, t)
            q = (t + NBUF - 1) % NBUF
            drain_gathers(q)
            start_store(r - 1, q)

        last = n_blk - 1
        drain_gathers(last % NBUF)
        start_store(last, last % NBUF)
        for p in range(NBUF):
            wait_store(p)

    return emb


def _tr_call(B, L, D, TS):
    M = L * D // 128

    def body(in_ref, out_ref):
        for t in range(TS // 128):
            out_ref[0, :, t * 128:(t + 1) * 128] = (
                in_ref[t * 128:(t + 1) * 128, :].T)

    return pl.pallas_call(
        body,
        grid=(M, B // TS),
        in_specs=[pl.BlockSpec((TS, 4 * D),
                               lambda m, i: (m * (B // TS) + i, 0))],
        out_specs=pl.BlockSpec((1, 4 * D, TS), lambda m, i: (m, 0, i)),
        out_shape=jax.ShapeDtypeStruct((M, 4 * D, B), jnp.float32),
    )


def kernel(indices, table):
    B, L = indices.shape
    V, D = table.shape
    N = B * L
    M = L * D // 128
    info = plsc.get_sparse_core_info()
    NC, NS = info.num_cores, info.num_subcores
    idx2d = (indices.reshape(B, M, 4).transpose(1, 0, 2)
             .reshape(N // 128, 128).astype(jnp.int32))
    g = _emb_call(N, D, NC, NS)(idx2d, table)
    g2 = g.reshape(N * D // 128, 128)
    t3 = _tr_call(B, L, D, 1024)(g2)
    return t3.reshape(L, D, B).transpose(2, 0, 1)


# speedup vs baseline: 2.1778x; 2.1778x over previous
import functools

import jax
import jax.numpy as jnp
from jax import lax
from jax.experimental import pallas as pl
from jax.experimental.pallas import tpu as pltpu
from jax.experimental.pallas import tpu_sc as plsc


def _emb_call(N, D, NC, NS):
    NW = NC * NS
    n_per_w = N // NW
    C = 128
    K = 8
    BLK = K * C
    NBUF = 3
    n_blk = n_per_w // BLK
    assert n_per_w % BLK == 0 and n_blk >= NBUF + 1

    mesh = plsc.VectorSubcoreMesh(core_axis_name="c", subcore_axis_name="s",
                                  num_cores=NC, num_subcores=NS)

    @functools.partial(
        pl.kernel,
        out_type=jax.ShapeDtypeStruct((N, D), jnp.float32),
        mesh=mesh,
        scratch_types=[
            pltpu.VMEM((NBUF, K, C), jnp.int32),
            pltpu.VMEM((NBUF, BLK, D), jnp.float32),
            [pltpu.SemaphoreType.DMA] * NBUF,
            [pltpu.SemaphoreType.DMA] * NBUF,
        ],
        compiler_params=pltpu.CompilerParams(use_tc_tiling_on_sc=False),
    )
    def emb(idx_hbm, table_hbm, out_hbm, idx_v, rows_v, gsems, osems):
        wid = lax.axis_index("s") * NC + lax.axis_index("c")
        base = wid * n_per_w

        def fire(i, p):
            b = pl.multiple_of(base + i * BLK, BLK)
            pltpu.sync_copy(idx_hbm.at[pl.ds(pl.multiple_of(b // C, K), K)],
                            idx_v.at[p])
            for j in range(K):
                pltpu.async_copy(table_hbm.at[idx_v.at[p, j]],
                                 rows_v.at[p, pl.ds(j * C, C)], gsems[p])

        def drain_gathers(p):
            pltpu.make_async_copy(out_hbm.at[pl.ds(0, BLK)], rows_v.at[p],
                                  gsems[p]).wait()

        def start_store(i, p):
            b = pl.multiple_of(base + i * BLK, BLK)
            pltpu.async_copy(rows_v.at[p], out_hbm.at[pl.ds(b, BLK)],
                             osems[p])

        def wait_store(p):
            pltpu.make_async_copy(out_hbm.at[pl.ds(0, BLK)], rows_v.at[p],
                                  osems[p]).wait()

        fire(0, 0)
        for i in range(1, NBUF):
            fire(i, i)
            drain_gathers(i - 1)
            start_store(i - 1, i - 1)

        n_loop = (n_blk - NBUF) // NBUF

        def body(g, carry):
            for t in range(NBUF):
                i = g * NBUF + t
                wait_store(t)
                fire(i, t)
                q = (t + NBUF - 1) % NBUF
                drain_gathers(q)
                start_store(i - 1, q)
            return carry

        lax.fori_loop(1, n_loop + 1, body, 0)

        for r in range(NBUF * (n_loop + 1), n_blk):
            t = r % NBUF
            wait_store(t)
            fire(r, t)
            q = (t + NBUF - 1) % NBUF
            drain_gathers(q)
            start_store(r - 1, q)

        last = n_blk - 1
        drain_gathers(last % NBUF)
        start_store(last, last % NBUF)
        for p in range(NBUF):
            wait_store(p)

    return emb


def _tr_call(B, L, D, NB):
    M = L // 4
    RB = M * NB

    def body(in_ref, out_ref):
        x3 = in_ref[...].reshape(NB, M, 4 * D)
        y = jnp.transpose(x3, (1, 0, 2))
        out_ref[...] = jnp.transpose(y, (0, 2, 1))

    return pl.pallas_call(
        body,
        grid=(B // NB,),
        in_specs=[pl.BlockSpec((RB, 4 * D), lambda i: (i, 0))],
        out_specs=pl.BlockSpec((M, 4 * D, NB), lambda i: (0, 0, i)),
        out_shape=jax.ShapeDtypeStruct((M, 4 * D, B), jnp.float32),
    )


def kernel(indices, table):
    B, L = indices.shape
    V, D = table.shape
    N = B * L
    info = plsc.get_sparse_core_info()
    NC, NS = info.num_cores, info.num_subcores
    idx2d = indices.reshape(N // 128, 128).astype(jnp.int32)
    g = _emb_call(N, D, NC, NS)(idx2d, table)
    g2 = g.reshape(N * D // 128, 128)
    t3 = _tr_call(B, L, D, 128)(g2)
    return t3.reshape(L, D, B).transpose(2, 0, 1)


# speedup vs baseline: 2.2386x; 1.0279x over previous
import functools

import jax
import jax.numpy as jnp
from jax import lax
from jax.experimental import pallas as pl
from jax.experimental.pallas import tpu as pltpu
from jax.experimental.pallas import tpu_sc as plsc


def _emb_call(N, D, NC, NS):
    NW = NC * NS
    n_per_w = N // NW
    C = 128
    K = 8
    BLK = K * C
    NBUF = 3
    n_blk = n_per_w // BLK
    assert n_per_w % BLK == 0 and n_blk >= NBUF + 1

    mesh = plsc.VectorSubcoreMesh(core_axis_name="c", subcore_axis_name="s",
                                  num_cores=NC, num_subcores=NS)

    @functools.partial(
        pl.kernel,
        out_type=jax.ShapeDtypeStruct((N, D), jnp.float32),
        mesh=mesh,
        scratch_types=[
            pltpu.VMEM((NBUF, K, C), jnp.int32),
            pltpu.VMEM((NBUF, BLK, D), jnp.float32),
            [pltpu.SemaphoreType.DMA] * NBUF,
            [pltpu.SemaphoreType.DMA] * NBUF,
        ],
        compiler_params=pltpu.CompilerParams(use_tc_tiling_on_sc=False),
    )
    def emb(idx_hbm, table_hbm, out_hbm, idx_v, rows_v, gsems, osems):
        wid = lax.axis_index("s") * NC + lax.axis_index("c")
        base = wid * n_per_w

        def fire(i, p):
            b = pl.multiple_of(base + i * BLK, BLK)
            pltpu.sync_copy(idx_hbm.at[pl.ds(pl.multiple_of(b // C, K), K)],
                            idx_v.at[p])
            for j in range(K):
                pltpu.async_copy(table_hbm.at[idx_v.at[p, j]],
                                 rows_v.at[p, pl.ds(j * C, C)], gsems[p])

        def drain_gathers(p):
            pltpu.make_async_copy(out_hbm.at[pl.ds(0, BLK)], rows_v.at[p],
                                  gsems[p]).wait()

        def start_store(i, p):
            b = pl.multiple_of(base + i * BLK, BLK)
            pltpu.async_copy(rows_v.at[p], out_hbm.at[pl.ds(b, BLK)],
                             osems[p])

        def wait_store(p):
            pltpu.make_async_copy(out_hbm.at[pl.ds(0, BLK)], rows_v.at[p],
                                  osems[p]).wait()

        fire(0, 0)
        for i in range(1, NBUF):
            fire(i, i)
            drain_gathers(i - 1)
            start_store(i - 1, i - 1)

        n_loop = (n_blk - NBUF) // NBUF

        def body(g, carry):
            for t in range(NBUF):
                i = g * NBUF + t
                wait_store(t)
                fire(i, t)
                q = (t + NBUF - 1) % NBUF
                drain_gathers(q)
                start_store(i - 1, q)
            return carry

        lax.fori_loop(1, n_loop + 1, body, 0)

        for r in range(NBUF * (n_loop + 1), n_blk):
            t = r % NBUF
            wait_store(t)
            fire(r, t)
            q = (t + NBUF - 1) % NBUF
            drain_gathers(q)
            start_store(r - 1, q)

        last = n_blk - 1
        drain_gathers(last % NBUF)
        start_store(last, last % NBUF)
        for p in range(NBUF):
            wait_store(p)

    return emb


def _tr_call(B, L, D, NB):
    M = L // 4
    RB = M * NB

    def body(in_ref, out_ref):
        x3 = in_ref[...].reshape(NB, M, 4 * D)
        y = jnp.transpose(x3, (1, 0, 2))
        out_ref[...] = jnp.transpose(y, (0, 2, 1))

    return pl.pallas_call(
        body,
        grid=(B // NB,),
        in_specs=[pl.BlockSpec((RB, 4 * D), lambda i: (i, 0))],
        out_specs=pl.BlockSpec((M, 4 * D, NB), lambda i: (0, 0, i)),
        out_shape=jax.ShapeDtypeStruct((M, 4 * D, B), jnp.float32),
    )


def kernel(indices, table):
    B, L = indices.shape
    V, D = table.shape
    N = B * L
    info = plsc.get_sparse_core_info()
    NC, NS = info.num_cores, info.num_subcores
    idx2d = indices.reshape(N // 128, 128).astype(jnp.int32)
    g = _emb_call(N, D, NC, NS)(idx2d, table)
    g2 = g.reshape(N * D // 128, 128)
    t3 = _tr_call(B, L, D, 256)(g2)
    return t3.reshape(L, D, B).transpose(2, 0, 1)


# speedup vs baseline: 2.2597x; 1.0094x over previous
import functools

import jax
import jax.numpy as jnp
from jax import lax
from jax.experimental import pallas as pl
from jax.experimental.pallas import tpu as pltpu
from jax.experimental.pallas import tpu_sc as plsc


def _emb_call(N, D, NC, NS):
    NW = NC * NS
    n_per_w = N // NW
    C = 128
    K = 8
    BLK = K * C
    NBUF = 3
    n_blk = n_per_w // BLK
    assert n_per_w % BLK == 0 and n_blk >= NBUF + 1

    mesh = plsc.VectorSubcoreMesh(core_axis_name="c", subcore_axis_name="s",
                                  num_cores=NC, num_subcores=NS)

    @functools.partial(
        pl.kernel,
        out_type=jax.ShapeDtypeStruct((N, D), jnp.float32),
        mesh=mesh,
        scratch_types=[
            pltpu.VMEM((NBUF, K, C), jnp.int32),
            pltpu.VMEM((NBUF, BLK, D), jnp.float32),
            [pltpu.SemaphoreType.DMA] * NBUF,
            [pltpu.SemaphoreType.DMA] * NBUF,
        ],
        compiler_params=pltpu.CompilerParams(use_tc_tiling_on_sc=False),
    )
    def emb(idx_hbm, table_hbm, out_hbm, idx_v, rows_v, gsems, osems):
        wid = lax.axis_index("s") * NC + lax.axis_index("c")
        base = wid * n_per_w

        def fire(i, p):
            b = pl.multiple_of(base + i * BLK, BLK)
            pltpu.sync_copy(idx_hbm.at[pl.ds(pl.multiple_of(b // C, K), K)],
                            idx_v.at[p])
            for j in range(K):
                pltpu.async_copy(table_hbm.at[idx_v.at[p, j]],
                                 rows_v.at[p, pl.ds(j * C, C)], gsems[p])

        def drain_gathers(p):
            pltpu.make_async_copy(out_hbm.at[pl.ds(0, BLK)], rows_v.at[p],
                                  gsems[p]).wait()

        def start_store(i, p):
            b = pl.multiple_of(base + i * BLK, BLK)
            pltpu.async_copy(rows_v.at[p], out_hbm.at[pl.ds(b, BLK)],
                             osems[p])

        def wait_store(p):
            pltpu.make_async_copy(out_hbm.at[pl.ds(0, BLK)], rows_v.at[p],
                                  osems[p]).wait()

        fire(0, 0)
        for i in range(1, NBUF):
            fire(i, i)
            drain_gathers(i - 1)
            start_store(i - 1, i - 1)

        n_loop = (n_blk - NBUF) // NBUF

        def body(g, carry):
            for t in range(NBUF):
                i = g * NBUF + t
                wait_store(t)
                fire(i, t)
                q = (t + NBUF - 1) % NBUF
                drain_gathers(q)
                start_store(i - 1, q)
            return carry

        lax.fori_loop(1, n_loop + 1, body, 0)

        for r in range(NBUF * (n_loop + 1), n_blk):
            t = r % NBUF
            wait_store(t)
            fire(r, t)
            q = (t + NBUF - 1) % NBUF
            drain_gathers(q)
            start_store(r - 1, q)

        last = n_blk - 1
        drain_gathers(last % NBUF)
        start_store(last, last % NBUF)
        for p in range(NBUF):
            wait_store(p)

    return emb


def _tr_call(B, L, D, NB):
    M = L // 4
    RB = M * NB

    def body(in_ref, out_ref):
        x3 = in_ref[...].reshape(NB, M, 4 * D)
        y = jnp.transpose(x3, (1, 0, 2))
        out_ref[...] = jnp.transpose(y, (0, 2, 1))

    return pl.pallas_call(
        body,
        grid=(B // NB,),
        in_specs=[pl.BlockSpec((RB, 4 * D), lambda i: (i, 0))],
        out_specs=pl.BlockSpec((M, 4 * D, NB), lambda i: (0, 0, i)),
        out_shape=jax.ShapeDtypeStruct((M, 4 * D, B), jnp.float32),
    )


def kernel(indices, table):
    B, L = indices.shape
    V, D = table.shape
    N = B * L
    info = plsc.get_sparse_core_info()
    NC, NS = info.num_cores, info.num_subcores
    idx2d = indices.reshape(N // 128, 128).astype(jnp.int32)
    g = _emb_call(N, D, NC, NS)(idx2d, table)
    g2 = g.reshape(N * D // 128, 128)
    t3 = _tr_call(B, L, D, 512)(g2)
    return t3.reshape(L, D, B).transpose(2, 0, 1)
